# baseline (speedup 1.0000x reference)
import functools
import jax
import jax.numpy as jnp
from jax import lax
from jax.experimental import pallas as pl
from jax.experimental.pallas import tpu as pltpu
from jax.experimental.pallas import tpu_sc as plsc

B = 32
L = 100
NW = 32
DT = 320
DW = 384
D_IN = NW * DT
D_OUT = NW * DW
F_PAD = 640
FB = 32
NFB = F_PAD // FB
NJ = DT // 16


def _sc_body(xT_hbm, id_hbm, lvl_hbm, out_hbm,
             xb_v, idx_v, lvl_v, id_v, acc_v, sem0, sem1):
    cid = lax.axis_index("c")
    sid = lax.axis_index("s")
    wid = sid * 2 + cid
    off = 64 * (wid % 2)
    da = pl.multiple_of(wid * DT - off, 128)

    pltpu.sync_copy(lvl_hbm.at[:, pl.ds(da, DW)], lvl_v)
    pltpu.sync_copy(xT_hbm, xb_v)

    @plsc.parallel_loop(0, F_PAD * B // 16, unroll=8)
    def idx_body(i):
        v = xb_v[pl.ds(i * 16, 16)]
        idx_v[pl.ds(i * 16, 16)] = jnp.clip(
            (v * float(L)).astype(jnp.int32), 0, L - 1)

    zero = jnp.zeros((16,), jnp.float32)

    @plsc.parallel_loop(0, B * (DW // 16), unroll=8)
    def zero_body(i):
        acc_v[i // (DW // 16), pl.ds(16 * (i % (DW // 16)), 16)] = zero

    lane_iota = lax.broadcasted_iota(jnp.int32, (16,), 0)
    cols = [lane_iota + (off + 16 * j) for j in range(NJ)]
    sems = (sem0, sem1)

    def id_slice(fb):
        return id_hbm.at[pl.ds(fb * FB, FB), pl.ds(da, DW)]

    pltpu.async_copy(id_slice(0), id_v.at[0], sem0)
    pltpu.async_copy(id_slice(1), id_v.at[1], sem1)

    def fb_pair_body(g, _):
        for par in range(2):
            fb = 2 * g + par
            pltpu.make_async_copy(id_slice(fb), id_v.at[par], sems[par]).wait()

            def fl_body(fl, _):
                idrow = [id_v[par, fl, pl.ds(off + 16 * j, 16)]
                         for j in range(NJ)]

                @plsc.parallel_loop(0, B, unroll=8)
                def b_body(b):
                    pos = jnp.full((16,), (fb * FB + fl) * B + b, jnp.int32)
                    row = plsc.load_gather(idx_v, [pos])
                    for j in range(NJ):
                        lv = plsc.load_gather(lvl_v, [row, cols[j]])
                        plsc.addupdate(
                            acc_v.at[b, pl.ds(off + 16 * j, 16)],
                            lv * idrow[j])

                return 0

            lax.fori_loop(0, FB, fl_body, 0)

            @pl.when(fb + 2 < NFB)
            def _():
                pltpu.async_copy(id_slice(fb + 2), id_v.at[par], sems[par])

        return 0

    lax.fori_loop(0, NFB // 2, fb_pair_body, 0)

    one = jnp.full((16,), 1.0, jnp.float32)
    neg = jnp.full((16,), -1.0, jnp.float32)

    @plsc.parallel_loop(0, B * (DW // 16), unroll=8)
    def q_body(i):
        b = i // (DW // 16)
        j = i % (DW // 16)
        s = acc_v[b, pl.ds(16 * j, 16)]
        acc_v[b, pl.ds(16 * j, 16)] = jnp.where(s > 0, one, neg)

    pltpu.sync_copy(acc_v, out_hbm.at[:, pl.ds(wid * DW, DW)])


def _classify_body(q_ref, cw_ref, out_ref):
    out_ref[...] = jax.lax.dot_general(
        q_ref[...], cw_ref[...],
        (((1,), (1,)), ((), ())),
        preferred_element_type=jnp.float32,
    )


def kernel(x, id_weight, level_weight, classify_weight):
    F = x.shape[1]
    D = level_weight.shape[1]
    C = classify_weight.shape[0]
    xT = jnp.pad(x.T, ((0, F_PAD - F), (0, 0))).reshape(-1)
    id_p = jnp.pad(id_weight, ((0, F_PAD - F), (0, D_IN - D)))
    lvl_p = jnp.pad(level_weight, ((0, 0), (0, D_IN - D)))

    cw_r = jnp.pad(classify_weight, ((0, 0), (0, D_IN - D))).reshape(C, NW, DT)
    cw_e = jnp.pad(cw_r, ((0, 0), (0, 0), (0, DW - DT)))
    cw_o = jnp.pad(cw_r, ((0, 0), (0, 0), (DW - DT, 0)))
    odd = (jnp.arange(NW) % 2) == 1
    cw_q = jnp.where(odd[None, :, None], cw_o, cw_e).reshape(C, D_OUT)

    mesh = plsc.VectorSubcoreMesh(
        core_axis_name="c", subcore_axis_name="s", num_cores=2, num_subcores=16)
    sc = functools.partial(
        pl.kernel,
        out_type=jax.ShapeDtypeStruct((B, D_OUT), jnp.float32),
        mesh=mesh,
        compiler_params=pltpu.CompilerParams(needs_layout_passes=False),
        scratch_types=[
            pltpu.VMEM((F_PAD * B,), jnp.float32),
            pltpu.VMEM((F_PAD * B,), jnp.int32),
            pltpu.VMEM((L, DW), jnp.float32),
            pltpu.VMEM((2, FB, DW), jnp.float32),
            pltpu.VMEM((B, DW), jnp.float32),
            pltpu.SemaphoreType.DMA,
            pltpu.SemaphoreType.DMA,
        ],
    )(_sc_body)
    q = sc(xT, id_p, lvl_p)

    logit = pl.pallas_call(
        _classify_body,
        in_specs=[
            pl.BlockSpec((B, D_OUT), lambda: (0, 0)),
            pl.BlockSpec((C, D_OUT), lambda: (0, 0)),
        ],
        out_specs=pl.BlockSpec((B, C), lambda: (0, 0)),
        out_shape=jax.ShapeDtypeStruct((B, C), jnp.float32),
    )(q, cw_q)
    return logit


# speedup vs baseline: 1.0625x; 1.0625x over previous
import functools
import jax
import jax.numpy as jnp
from jax import lax
from jax.experimental import pallas as pl
from jax.experimental.pallas import tpu as pltpu
from jax.experimental.pallas import tpu_sc as plsc

B = 32
L = 100
NW = 32
DT = 320
DW = 384
D_IN = NW * DT
D_OUT = NW * DW
F_PAD = 640
FB = 32
NFB = F_PAD // FB
NJ = DT // 16


def _sc_body(xT_hbm, id_hbm, lvl_hbm, out_hbm,
             xb_v, idx_v, lvl_v, id_v, acc_v, sem0, sem1):
    cid = lax.axis_index("c")
    sid = lax.axis_index("s")
    wid = sid * 2 + cid
    off = 64 * (wid % 2)
    da = pl.multiple_of(wid * DT - off, 128)

    pltpu.sync_copy(lvl_hbm.at[:, pl.ds(da, DW)], lvl_v)
    pltpu.sync_copy(xT_hbm, xb_v)

    @plsc.parallel_loop(0, F_PAD * B // 16, unroll=8)
    def idx_body(i):
        v = xb_v[pl.ds(i * 16, 16)]
        idx_v[pl.ds(i * 16, 16)] = jnp.clip(
            (v * float(L)).astype(jnp.int32), 0, L - 1)

    zero = jnp.zeros((16,), jnp.float32)

    @plsc.parallel_loop(0, B * (DW // 16), unroll=8)
    def zero_body(i):
        acc_v[i // (DW // 16), pl.ds(16 * (i % (DW // 16)), 16)] = zero

    lane_iota = lax.broadcasted_iota(jnp.int32, (16,), 0)
    cols = [lane_iota + (off + 16 * j) for j in range(NJ)]
    sems = (sem0, sem1)

    def id_slice(fb):
        return id_hbm.at[pl.ds(fb * FB, FB), pl.ds(da, DW)]

    pltpu.async_copy(id_slice(0), id_v.at[0], sem0)
    pltpu.async_copy(id_slice(1), id_v.at[1], sem1)

    def fb_pair_body(g, _):
        for par in range(2):
            fb = 2 * g + par
            pltpu.make_async_copy(id_slice(fb), id_v.at[par], sems[par]).wait()

            def fl_body(fl, _):
                idrow = [id_v[par, fl, pl.ds(off + 16 * j, 16)]
                         for j in range(NJ)]

                @plsc.parallel_loop(0, B, unroll=4)
                def b_body(b):
                    pos = jnp.full((16,), (fb * FB + fl) * B + b, jnp.int32)
                    row = plsc.load_gather(idx_v, [pos])
                    for j in range(NJ):
                        lv = plsc.load_gather(lvl_v, [row, cols[j]])
                        plsc.addupdate(
                            acc_v.at[b, pl.ds(off + 16 * j, 16)],
                            lv * idrow[j])

                return 0

            lax.fori_loop(0, FB, fl_body, 0)

            @pl.when(fb + 2 < NFB)
            def _():
                pltpu.async_copy(id_slice(fb + 2), id_v.at[par], sems[par])

        return 0

    lax.fori_loop(0, NFB // 2, fb_pair_body, 0)

    one = jnp.full((16,), 1.0, jnp.float32)
    neg = jnp.full((16,), -1.0, jnp.float32)

    @plsc.parallel_loop(0, B * (DW // 16), unroll=8)
    def q_body(i):
        b = i // (DW // 16)
        j = i % (DW // 16)
        s = acc_v[b, pl.ds(16 * j, 16)]
        acc_v[b, pl.ds(16 * j, 16)] = jnp.where(s > 0, one, neg)

    pltpu.sync_copy(acc_v, out_hbm.at[:, pl.ds(wid * DW, DW)])


def _classify_body(q_ref, cw_ref, out_ref):
    out_ref[...] = jax.lax.dot_general(
        q_ref[...], cw_ref[...],
        (((1,), (1,)), ((), ())),
        preferred_element_type=jnp.float32,
    )


def kernel(x, id_weight, level_weight, classify_weight):
    F = x.shape[1]
    D = level_weight.shape[1]
    C = classify_weight.shape[0]
    xT = jnp.pad(x.T, ((0, F_PAD - F), (0, 0))).reshape(-1)
    id_p = jnp.pad(id_weight, ((0, F_PAD - F), (0, D_IN - D)))
    lvl_p = jnp.pad(level_weight, ((0, 0), (0, D_IN - D)))

    cw_r = jnp.pad(classify_weight, ((0, 0), (0, D_IN - D))).reshape(C, NW, DT)
    cw_e = jnp.pad(cw_r, ((0, 0), (0, 0), (0, DW - DT)))
    cw_o = jnp.pad(cw_r, ((0, 0), (0, 0), (DW - DT, 0)))
    odd = (jnp.arange(NW) % 2) == 1
    cw_q = jnp.where(odd[None, :, None], cw_o, cw_e).reshape(C, D_OUT)

    mesh = plsc.VectorSubcoreMesh(
        core_axis_name="c", subcore_axis_name="s", num_cores=2, num_subcores=16)
    sc = functools.partial(
        pl.kernel,
        out_type=jax.ShapeDtypeStruct((B, D_OUT), jnp.float32),
        mesh=mesh,
        compiler_params=pltpu.CompilerParams(needs_layout_passes=False),
        scratch_types=[
            pltpu.VMEM((F_PAD * B,), jnp.float32),
            pltpu.VMEM((F_PAD * B,), jnp.int32),
            pltpu.VMEM((L, DW), jnp.float32),
            pltpu.VMEM((2, FB, DW), jnp.float32),
            pltpu.VMEM((B, DW), jnp.float32),
            pltpu.SemaphoreType.DMA,
            pltpu.SemaphoreType.DMA,
        ],
    )(_sc_body)
    q = sc(xT, id_p, lvl_p)

    logit = pl.pallas_call(
        _classify_body,
        in_specs=[
            pl.BlockSpec((B, D_OUT), lambda: (0, 0)),
            pl.BlockSpec((C, D_OUT), lambda: (0, 0)),
        ],
        out_specs=pl.BlockSpec((B, C), lambda: (0, 0)),
        out_shape=jax.ShapeDtypeStruct((B, C), jnp.float32),
    )(q, cw_q)
    return logit
